# baseline (speedup 1.0000x reference)
import functools

import jax
import jax.numpy as jnp
from jax import lax
from jax.experimental import pallas as pl
from jax.experimental.pallas import tpu as pltpu
from jax.experimental.pallas import tpu_sc as plsc

NC, NS, L = 2, 16, 16
NW = NC * NS


def _make_sc_kernel(B, S, D, V):
    ROWS = B * S
    CHUNK = S
    RPW = ROWS // NW
    NCHUNK = RPW // CHUNK
    GROUPS = (CHUNK * D) // L

    mesh = plsc.VectorSubcoreMesh(
        core_axis_name="c", subcore_axis_name="s", num_cores=NC, num_subcores=NS
    )

    @functools.partial(
        pl.kernel,
        out_type=jax.ShapeDtypeStruct((ROWS, D), jnp.float32),
        mesh=mesh,
        scratch_types=[
            pltpu.VMEM((S, D), jnp.float32),
            pltpu.VMEM((CHUNK,), jnp.int32),
            pltpu.VMEM((CHUNK,), jnp.int32),
            pltpu.VMEM((CHUNK, D), jnp.float32),
            pltpu.VMEM((CHUNK, D), jnp.float32),
            pltpu.SemaphoreType.DMA,
            pltpu.SemaphoreType.DMA,
        ],
    )
    def sc_kernel(ids_hbm, seg_hbm, tok_hbm, segtab_hbm, pe_hbm, out_hbm,
                  pe_v, idx_v, sidx_v, buf_a, buf_b, sem_a, sem_b):
        wid = lax.axis_index("s") * NC + lax.axis_index("c")
        pltpu.sync_copy(pe_hbm, pe_v)

        def chunk_body(ci, carry):
            base = wid * RPW + ci * CHUNK
            pltpu.sync_copy(ids_hbm.at[pl.ds(base, CHUNK)], idx_v)
            cp_a = pltpu.async_copy(tok_hbm.at[idx_v], buf_a, sem_a)
            pltpu.sync_copy(seg_hbm.at[pl.ds(base, CHUNK)], sidx_v)
            cp_b = pltpu.async_copy(segtab_hbm.at[sidx_v], buf_b, sem_b)
            cp_a.wait()
            cp_b.wait()

            def row_body(j, carry2):
                for k in range(D // L):
                    sl = pl.ds(k * L, L)
                    buf_a[j, sl] = buf_a[j, sl] + buf_b[j, sl] + pe_v[j, sl]
                return carry2

            lax.fori_loop(0, CHUNK, row_body, 0, unroll=False)
            pltpu.sync_copy(buf_a, out_hbm.at[pl.ds(base, CHUNK)])
            return carry

        lax.fori_loop(0, NCHUNK, chunk_body, 0, unroll=False)

    return sc_kernel


def kernel(ids, segment_label, token_table, segment_table, pe):
    B, S = ids.shape
    V, D = token_table.shape
    ids_f = ids.reshape(-1).astype(jnp.int32)
    seg_f = segment_label.reshape(-1).astype(jnp.int32)
    pe2 = pe.reshape(S, D).astype(jnp.float32)
    sc = _make_sc_kernel(B, S, D, V)
    out = sc(ids_f, seg_f, token_table, segment_table, pe2)
    return out.reshape(B, S, D)


# speedup vs baseline: 8.9880x; 8.9880x over previous
import functools

import jax
import jax.numpy as jnp
from jax import lax
from jax.experimental import pallas as pl
from jax.experimental.pallas import tpu as pltpu
from jax.experimental.pallas import tpu_sc as plsc

NC, NS, L = 2, 16, 16
NW = NC * NS


def _make_sc_kernel(B, S, D, V, NSEG):
    ROWS = B * S
    CHUNK = S
    RPW = ROWS // NW
    NCHUNK = RPW // CHUNK
    NPAIR = NCHUNK // 2
    KG = D // L

    mesh = plsc.VectorSubcoreMesh(
        core_axis_name="c", subcore_axis_name="s", num_cores=NC, num_subcores=NS
    )

    @functools.partial(
        pl.kernel,
        out_type=jax.ShapeDtypeStruct((ROWS, D), jnp.float32),
        mesh=mesh,
        scratch_types=[
            pltpu.VMEM((NSEG * S, D), jnp.float32),
            pltpu.VMEM((NSEG, D), jnp.float32),
            pltpu.VMEM((RPW,), jnp.int32),
            pltpu.VMEM((RPW,), jnp.int32),
            pltpu.VMEM((CHUNK, D), jnp.float32),
            pltpu.VMEM((CHUNK, D), jnp.float32),
            pltpu.VMEM((CHUNK, D), jnp.float32),
            pltpu.VMEM((CHUNK, D), jnp.float32),
            pltpu.SemaphoreType.DMA,
            pltpu.SemaphoreType.DMA,
            pltpu.SemaphoreType.DMA,
            pltpu.SemaphoreType.DMA,
        ],
    )
    def sc_kernel(ids_hbm, seg_hbm, tok_hbm, segtab_hbm, pe_hbm, out_hbm,
                  comb, segtab_v, idx_all, sidx_all,
                  gbuf_a, gbuf_b, obuf_a, obuf_b,
                  gsem_a, gsem_b, osem_a, osem_b):
        wid = lax.axis_index("s") * NC + lax.axis_index("c")
        tbase = wid * RPW

        pltpu.sync_copy(ids_hbm.at[pl.ds(tbase, RPW)], idx_all)
        pltpu.sync_copy(seg_hbm.at[pl.ds(tbase, RPW)], sidx_all)
        pltpu.sync_copy(segtab_hbm, segtab_v)
        pltpu.sync_copy(pe_hbm, comb.at[pl.ds(0, S)])

        def build_row(j, carry):
            for k in range(KG):
                sl = pl.ds(k * L, L)
                pe_k = comb[j, sl]
                for g in range(1, NSEG):
                    comb[g * S + j, sl] = pe_k + segtab_v[g, sl]
                comb[j, sl] = pe_k + segtab_v[0, sl]
            return carry

        lax.fori_loop(0, S, build_row, 0, unroll=False)

        def gather(c, gbuf, gsem):
            return pltpu.async_copy(
                tok_hbm.at[idx_all.at[pl.ds(c * CHUNK, CHUNK)]], gbuf, gsem)

        def compute(c, gbuf, obuf):
            def grp_body(jg, carry2):
                j0 = jg * L
                segv = sidx_all[pl.ds(c * CHUNK + j0, L)]
                for l in range(L):
                    crow = segv[l] * S + (j0 + l)
                    for k in range(KG):
                        sl = pl.ds(k * L, L)
                        obuf[j0 + l, sl] = gbuf[j0 + l, sl] + comb[crow, sl]
                return carry2
            lax.fori_loop(0, CHUNK // L, grp_body, 0, unroll=False)

        def store(c, obuf, osem):
            return pltpu.async_copy(
                obuf, out_hbm.at[pl.ds(tbase + c * CHUNK, CHUNK)], osem)

        gather(0, gbuf_a, gsem_a)

        def pair_body(i, carry):
            c0 = 2 * i
            c1 = c0 + 1
            gather(c1, gbuf_b, gsem_b)
            pltpu.make_async_copy(tok_hbm.at[idx_all.at[pl.ds(0, CHUNK)]],
                                  gbuf_a, gsem_a).wait()

            @pl.when(i > 0)
            def _():
                pltpu.make_async_copy(obuf_a, out_hbm.at[pl.ds(0, CHUNK)],
                                      osem_a).wait()

            compute(c0, gbuf_a, obuf_a)
            store(c0, obuf_a, osem_a)

            @pl.when(i < NPAIR - 1)
            def _():
                gather(c0 + 2, gbuf_a, gsem_a)

            pltpu.make_async_copy(tok_hbm.at[idx_all.at[pl.ds(0, CHUNK)]],
                                  gbuf_b, gsem_b).wait()

            @pl.when(i > 0)
            def _():
                pltpu.make_async_copy(obuf_b, out_hbm.at[pl.ds(0, CHUNK)],
                                      osem_b).wait()

            compute(c1, gbuf_b, obuf_b)
            store(c1, obuf_b, osem_b)
            return carry

        lax.fori_loop(0, NPAIR, pair_body, 0, unroll=False)

        pltpu.make_async_copy(obuf_a, out_hbm.at[pl.ds(0, CHUNK)], osem_a).wait()
        pltpu.make_async_copy(obuf_b, out_hbm.at[pl.ds(0, CHUNK)], osem_b).wait()

    return sc_kernel


def kernel(ids, segment_label, token_table, segment_table, pe):
    B, S = ids.shape
    V, D = token_table.shape
    NSEG = segment_table.shape[0]
    ids_f = ids.reshape(-1).astype(jnp.int32)
    seg_f = segment_label.reshape(-1).astype(jnp.int32)
    pe2 = pe.reshape(S, D).astype(jnp.float32)
    sc = _make_sc_kernel(B, S, D, V, NSEG)
    out = sc(ids_f, seg_f, token_table, segment_table, pe2)
    return out.reshape(B, S, D)


# speedup vs baseline: 12.7800x; 1.4219x over previous
import functools

import jax
import jax.numpy as jnp
from jax import lax
from jax.experimental import pallas as pl
from jax.experimental.pallas import tpu as pltpu
from jax.experimental.pallas import tpu_sc as plsc

NC, NS, L = 2, 16, 16
NW = NC * NS
NBUF = 4


def _build_comb(segment_table, pe2):
    NSEG, D = segment_table.shape
    S = pe2.shape[0]

    def comb_tc(seg_ref, pe_ref, out_ref):
        for g in range(NSEG):
            out_ref[g * S:(g + 1) * S, :] = (
                pe_ref[...] + seg_ref[g, :][None, :])

    return pl.pallas_call(
        comb_tc,
        out_shape=jax.ShapeDtypeStruct((NSEG * S, D), jnp.float32),
    )(segment_table, pe2)


def _make_sc_kernel(B, S, D, NSEG):
    ROWS = B * S
    CHUNK = S
    RPW = ROWS // NW
    NCHUNK = RPW // CHUNK
    NITER = NCHUNK // NBUF

    mesh = plsc.VectorSubcoreMesh(
        core_axis_name="c", subcore_axis_name="s", num_cores=NC, num_subcores=NS
    )

    @functools.partial(
        pl.kernel,
        out_type=jax.ShapeDtypeStruct((ROWS, D), jnp.float32),
        mesh=mesh,
        scratch_types=[
            pltpu.VMEM((RPW,), jnp.int32),
            pltpu.VMEM((RPW,), jnp.int32),
            pltpu.VMEM((NBUF * CHUNK,), jnp.int32),
            [pltpu.VMEM((CHUNK, D), jnp.float32) for _ in range(NBUF)],
            [pltpu.SemaphoreType.DMA for _ in range(NBUF)],
            [pltpu.SemaphoreType.DMA for _ in range(NBUF)],
        ],
    )
    def sc_kernel(ids_hbm, seg_hbm, tok_hbm, comb_hbm, out_hbm,
                  idx_all, sidx_all, crow, bufs, gsems, osems):
        wid = lax.axis_index("s") * NC + lax.axis_index("c")
        tbase = wid * RPW
        pltpu.sync_copy(ids_hbm.at[pl.ds(tbase, RPW)], idx_all)
        pltpu.sync_copy(seg_hbm.at[pl.ds(tbase, RPW)], sidx_all)

        def iter_body(i, carry):
            c0 = i * NBUF
            for k in range(NBUF):
                c = c0 + k

                @pl.when(i > 0)
                def _():
                    pltpu.make_async_copy(
                        bufs[k], out_hbm.at[pl.ds(0, CHUNK)], osems[k]).wait()

                for jg in range(CHUNK // L):
                    j0 = jg * L
                    segv = sidx_all[pl.ds(c * CHUNK + j0, L)]
                    crow[pl.ds(k * CHUNK + j0, L)] = (
                        segv * S + (j0 + lax.iota(jnp.int32, L)))
                pltpu.async_copy(
                    comb_hbm.at[crow.at[pl.ds(k * CHUNK, CHUNK)]],
                    bufs[k], gsems[k])
            for k in range(NBUF):
                c = c0 + k
                pltpu.make_async_copy(
                    comb_hbm.at[crow.at[pl.ds(k * CHUNK, CHUNK)]],
                    bufs[k], gsems[k]).wait()
                pltpu.async_copy(
                    tok_hbm.at[idx_all.at[pl.ds(c * CHUNK, CHUNK)]],
                    bufs[k], gsems[k], add=True)
            for k in range(NBUF):
                c = c0 + k
                pltpu.make_async_copy(
                    tok_hbm.at[idx_all.at[pl.ds(c * CHUNK, CHUNK)]],
                    bufs[k], gsems[k]).wait()
                pltpu.async_copy(
                    bufs[k], out_hbm.at[pl.ds(tbase + c * CHUNK, CHUNK)],
                    osems[k])
            return carry

        lax.fori_loop(0, NITER, iter_body, 0, unroll=False)
        for k in range(NBUF):
            pltpu.make_async_copy(
                bufs[k], out_hbm.at[pl.ds(0, CHUNK)], osems[k]).wait()

    return sc_kernel


def kernel(ids, segment_label, token_table, segment_table, pe):
    B, S = ids.shape
    V, D = token_table.shape
    NSEG = segment_table.shape[0]
    ids_f = ids.reshape(-1).astype(jnp.int32)
    seg_f = segment_label.reshape(-1).astype(jnp.int32)
    pe2 = pe.reshape(S, D).astype(jnp.float32)
    comb = _build_comb(segment_table.astype(jnp.float32), pe2)
    sc = _make_sc_kernel(B, S, D, NSEG)
    out = sc(ids_f, seg_f, token_table, comb)
    return out.reshape(B, S, D)


# speedup vs baseline: 25.8923x; 2.0260x over previous
import functools

import jax
import jax.numpy as jnp
from jax import lax
from jax.experimental import pallas as pl
from jax.experimental.pallas import tpu as pltpu
from jax.experimental.pallas import tpu_sc as plsc

NC, NS, L = 2, 16, 16
NW = NC * NS
NBUF = 4


def _build_comb(segment_table, pe2):
    NSEG, D = segment_table.shape
    S = pe2.shape[0]

    def comb_tc(seg_ref, pe_ref, out_ref):
        for g in range(NSEG):
            out_ref[g * S:(g + 1) * S, :] = (
                pe_ref[...] + seg_ref[g, :][None, :])

    return pl.pallas_call(
        comb_tc,
        out_shape=jax.ShapeDtypeStruct((NSEG * S, D), jnp.float32),
    )(segment_table, pe2)


def _make_sc_kernel(B, S, D, NSEG):
    ROWS = B * S
    CHUNK = S
    RPW = ROWS // NW
    NCHUNK = RPW // CHUNK
    NITER = NCHUNK // NBUF

    mesh = plsc.VectorSubcoreMesh(
        core_axis_name="c", subcore_axis_name="s", num_cores=NC, num_subcores=NS
    )

    @functools.partial(
        pl.kernel,
        out_type=jax.ShapeDtypeStruct((ROWS, D), jnp.float32),
        mesh=mesh,
        scratch_types=[
            pltpu.VMEM((RPW,), jnp.int32),
            pltpu.VMEM((RPW,), jnp.int32),
            pltpu.VMEM((NBUF * CHUNK,), jnp.int32),
            pltpu.VMEM_SHARED((NSEG * S, D), jnp.float32),
            [pltpu.VMEM((CHUNK, D), jnp.float32) for _ in range(NBUF)],
            [pltpu.SemaphoreType.DMA for _ in range(NBUF)],
            [pltpu.SemaphoreType.DMA for _ in range(NBUF)],
        ],
    )
    def sc_kernel(ids_hbm, seg_hbm, tok_hbm, comb_hbm, out_hbm,
                  idx_all, sidx_all, crow, comb_sh, bufs, gsems, osems):
        wid = lax.axis_index("s") * NC + lax.axis_index("c")
        tbase = wid * RPW
        pltpu.sync_copy(ids_hbm.at[pl.ds(tbase, RPW)], idx_all)
        pltpu.sync_copy(seg_hbm.at[pl.ds(tbase, RPW)], sidx_all)

        @pl.when(lax.axis_index("s") == 0)
        def _():
            pltpu.sync_copy(comb_hbm, comb_sh)

        plsc.subcore_barrier()

        def iter_body(i, carry):
            c0 = i * NBUF
            for k in range(NBUF):
                c = c0 + k

                @pl.when(i > 0)
                def _():
                    pltpu.make_async_copy(
                        bufs[k], out_hbm.at[pl.ds(0, CHUNK)], osems[k]).wait()

                for jg in range(CHUNK // L):
                    j0 = jg * L
                    segv = sidx_all[pl.ds(c * CHUNK + j0, L)]
                    crow[pl.ds(k * CHUNK + j0, L)] = (
                        segv * S + (j0 + lax.iota(jnp.int32, L)))
                pltpu.async_copy(
                    comb_sh.at[crow.at[pl.ds(k * CHUNK, CHUNK)]],
                    bufs[k], gsems[k])
            for k in range(NBUF):
                c = c0 + k
                pltpu.make_async_copy(
                    comb_sh.at[crow.at[pl.ds(k * CHUNK, CHUNK)]],
                    bufs[k], gsems[k]).wait()
                pltpu.async_copy(
                    tok_hbm.at[idx_all.at[pl.ds(c * CHUNK, CHUNK)]],
                    bufs[k], gsems[k], add=True)
            for k in range(NBUF):
                c = c0 + k
                pltpu.make_async_copy(
                    tok_hbm.at[idx_all.at[pl.ds(c * CHUNK, CHUNK)]],
                    bufs[k], gsems[k]).wait()
                pltpu.async_copy(
                    bufs[k], out_hbm.at[pl.ds(tbase + c * CHUNK, CHUNK)],
                    osems[k])
            return carry

        lax.fori_loop(0, NITER, iter_body, 0, unroll=False)
        for k in range(NBUF):
            pltpu.make_async_copy(
                bufs[k], out_hbm.at[pl.ds(0, CHUNK)], osems[k]).wait()

    return sc_kernel


def kernel(ids, segment_label, token_table, segment_table, pe):
    B, S = ids.shape
    V, D = token_table.shape
    NSEG = segment_table.shape[0]
    ids_f = ids.reshape(-1).astype(jnp.int32)
    seg_f = segment_label.reshape(-1).astype(jnp.int32)
    pe2 = pe.reshape(S, D).astype(jnp.float32)
    comb = _build_comb(segment_table.astype(jnp.float32), pe2)
    sc = _make_sc_kernel(B, S, D, NSEG)
    out = sc(ids_f, seg_f, token_table, comb)
    return out.reshape(B, S, D)
